# baseline (speedup 1.0000x reference)
import functools

import jax
import jax.numpy as jnp
from jax import lax
from jax.experimental import pallas as pl
from jax.experimental.pallas import tpu as pltpu
from jax.experimental.pallas import tpu_sc as plsc

K = 16
NBIN = 17
L = 32
P = 384 * 384
BS = 4
EXT = L + 2
NW = 32
PPW = P // NW
CH = 768
NCHUNK = PPW // CH
NGRP = CH // 16
ROWS = EXT * NBIN


def _sc_stage1(pred_hbm, t_hbm, ig_hbm, out_hbm, pbuf, tbuf, igbuf, acc,
               psem, tsem, isem):
    wid = lax.axis_index("s") * 2 + lax.axis_index("c")
    lane = lax.broadcasted_iota(jnp.int32, (16,), 0)
    nchunks = BS * NCHUNK

    @plsc.parallel_loop(0, BS * ROWS, unroll=8)
    def _zero(i):
        acc[pl.ds(i * 16, 16)] = jnp.zeros((16,), jnp.float32)

    def _start(chunk, buf):
        img = chunk // NCHUNK
        base = wid * PPW + (chunk % NCHUNK) * CH
        pltpu.make_async_copy(
            pred_hbm.at[img, :, pl.ds(base, CH)], pbuf.at[buf],
            psem.at[buf]).start()
        pltpu.make_async_copy(
            t_hbm.at[img, pl.ds(base, CH)], tbuf.at[buf],
            tsem.at[buf]).start()
        pltpu.make_async_copy(
            ig_hbm.at[img, pl.ds(base, CH)], igbuf.at[buf],
            isem.at[buf]).start()

    def _wait(buf):
        pltpu.make_async_copy(
            pred_hbm.at[0, :, pl.ds(0, CH)], pbuf.at[buf],
            psem.at[buf]).wait()
        pltpu.make_async_copy(
            t_hbm.at[0, pl.ds(0, CH)], tbuf.at[buf], tsem.at[buf]).wait()
        pltpu.make_async_copy(
            ig_hbm.at[0, pl.ds(0, CH)], igbuf.at[buf], isem.at[buf]).wait()

    def _compute(chunk, buf):
        img = chunk // NCHUNK
        img_base = img * (ROWS * 16)

        @plsc.parallel_loop(0, NGRP, unroll=2)
        def _group(g):
            o = g * 16
            t = tbuf[buf, pl.ds(o, 16)]
            ig = igbuf[buf, pl.ds(o, 16)]
            t_eff = jnp.where(ig == 0, t, 0)
            pv = [pbuf[buf, c, pl.ds(o, 16)] for c in range(L)]
            sq = [v * v for v in pv]
            while len(sq) > 1:
                sq = [sq[2 * i] + sq[2 * i + 1] for i in range(len(sq) // 2)]
            n2 = sq[0]
            xi = lax.bitcast_convert_type(n2, jnp.int32)
            yi = jnp.int32(0x5F3759DF) - (xi >> 1)
            y = lax.bitcast_convert_type(yi, jnp.float32)
            for _ in range(2):
                y = y * (1.5 - 0.5 * n2 * y * y)
            inv = y - 1e-6 * (y * y)
            psq = n2 * inv * inv
            bvec = img_base + t_eff * 16 + lane
            for c in range(L):
                plsc.addupdate_scatter(
                    acc, [bvec + (c * NBIN * 16)], pv[c] * inv)
            plsc.addupdate_scatter(
                acc, [bvec + (L * NBIN * 16)], jnp.ones((16,), jnp.float32))
            plsc.addupdate_scatter(
                acc, [bvec + ((L + 1) * NBIN * 16)], psq)

    _start(0, 0)

    def _pair(i, carry):
        for b in range(2):
            cur = 2 * i + b
            nxt = cur + 1

            @pl.when(nxt < nchunks)
            def _prefetch():
                _start(nxt, 1 - b)

            _wait(b)
            _compute(cur, b)
        return carry

    lax.fori_loop(0, nchunks // 2, _pair, 0)
    pltpu.sync_copy(acc, out_hbm.at[wid])


def _tc_stage2(part_ref, out_ref):
    i = pl.program_id(0)

    @pl.when(i == 0)
    def _init():
        out_ref[...] = jnp.zeros_like(out_ref)

    x = part_ref[0]
    cols = [jnp.sum(x[c * NBIN + 1:c * NBIN + 1 + K, :], axis=1,
                    keepdims=True) for c in range(EXT)]
    acc = jnp.concatenate(cols, axis=1)
    sums = acc[:, :L]
    cnt = acc[:, L:L + 1]
    sq = acc[:, L + 1:L + 2]
    present = cnt > 0.0
    pm = present.astype(jnp.float32)
    nf = jnp.sum(pm)
    cnt_safe = jnp.maximum(cnt, 1.0)
    ssum = jnp.sum(sums * sums, axis=1, keepdims=True)
    mse = (sq - ssum / cnt_safe) / (L * cnt_safe)
    pull_sum = jnp.sum(jnp.where(present, mse, 0.0))
    s_col = jnp.sum(sums, axis=1, keepdims=True) / cnt_safe
    onehot_cnt = (lax.broadcasted_iota(jnp.int32, (1, EXT), 1) == L
                  ).astype(jnp.float32)
    cnt_row = lax.dot_general(
        onehot_cnt, acc, (((1,), (1,)), ((), ())),
        preferred_element_type=jnp.float32)
    ones_row = jnp.ones((1, L), jnp.float32)
    s_row = lax.dot_general(
        ones_row, sums / cnt_safe, (((1,), (1,)), ((), ())),
        preferred_element_type=jnp.float32)
    pm_row = (cnt_row > 0.0).astype(jnp.float32)
    ds = s_row - s_col
    push_raw = jnp.sum(pm * pm_row * jnp.exp(-(ds * ds)))
    eps = 1e-6
    pull = jnp.where(nf > 0.0, pull_sum / (nf + eps), 0.0)
    push = jnp.where(nf > 1.0, push_raw / ((nf - 1.0) * nf + eps), 0.0)
    out_ref[...] += jnp.reshape(pull + 0.1 * push, (1, 1))


@jax.jit
def kernel(pred, target, ignore_position):
    predr = pred.reshape(BS, L, P)
    tr = target.reshape(BS, P).astype(jnp.int32)
    igr = ignore_position.reshape(BS, P).astype(jnp.int32)

    mesh = plsc.VectorSubcoreMesh(core_axis_name="c", subcore_axis_name="s")
    stage1 = functools.partial(
        pl.kernel,
        mesh=mesh,
        out_type=jax.ShapeDtypeStruct((NW, BS * ROWS * 16), jnp.float32),
        scratch_types=[
            pltpu.VMEM((2, L, CH), jnp.float32),
            pltpu.VMEM((2, CH), jnp.int32),
            pltpu.VMEM((2, CH), jnp.int32),
            pltpu.VMEM((BS * ROWS * 16,), jnp.float32),
            pltpu.SemaphoreType.DMA((2,)),
            pltpu.SemaphoreType.DMA((2,)),
            pltpu.SemaphoreType.DMA((2,)),
        ],
        compiler_params=pltpu.CompilerParams(needs_layout_passes=False),
    )(_sc_stage1)
    partials = stage1(predr, tr, igr)

    part = (partials.reshape(NW, BS, ROWS, 16)
            .transpose(1, 2, 0, 3).reshape(BS, ROWS, NW * 16))
    out = pl.pallas_call(
        _tc_stage2,
        grid=(BS,),
        in_specs=[pl.BlockSpec((1, ROWS, NW * 16), lambda i: (i, 0, 0))],
        out_specs=pl.BlockSpec((1, 1), lambda i: (0, 0)),
        out_shape=jax.ShapeDtypeStruct((1, 1), jnp.float32),
        compiler_params=pltpu.CompilerParams(
            dimension_semantics=("arbitrary",)),
    )(part)
    return out[0, 0]


# speedup vs baseline: 1.1635x; 1.1635x over previous
import functools

import jax
import jax.numpy as jnp
from jax import lax
from jax.experimental import pallas as pl
from jax.experimental.pallas import tpu as pltpu
from jax.experimental.pallas import tpu_sc as plsc

K = 16
NBIN = 17
L = 32
P = 384 * 384
BS = 4
EXT = L + 2
NW = 32
PPW = P // NW
CH = 768
NCHUNK = PPW // CH
NGRP = CH // 16
ROWS = EXT * NBIN


def _sc_stage1(pred_hbm, t_hbm, ig_hbm, out_hbm, pbuf, tbuf, igbuf, acc,
               psem, tsem, isem):
    wid = lax.axis_index("s") * 2 + lax.axis_index("c")
    lane = lax.broadcasted_iota(jnp.int32, (16,), 0)
    nchunks = BS * NCHUNK

    @plsc.parallel_loop(0, BS * ROWS, unroll=8)
    def _zero(i):
        acc[pl.ds(i * 16, 16)] = jnp.zeros((16,), jnp.float32)

    def _start(chunk, buf):
        img = chunk // NCHUNK
        base = wid * PPW + (chunk % NCHUNK) * CH
        pltpu.make_async_copy(
            pred_hbm.at[img, :, pl.ds(base, CH)], pbuf.at[buf],
            psem.at[buf]).start()
        pltpu.make_async_copy(
            t_hbm.at[img, pl.ds(base, CH)], tbuf.at[buf],
            tsem.at[buf]).start()
        pltpu.make_async_copy(
            ig_hbm.at[img, pl.ds(base, CH)], igbuf.at[buf],
            isem.at[buf]).start()

    def _wait(buf):
        pltpu.make_async_copy(
            pred_hbm.at[0, :, pl.ds(0, CH)], pbuf.at[buf],
            psem.at[buf]).wait()
        pltpu.make_async_copy(
            t_hbm.at[0, pl.ds(0, CH)], tbuf.at[buf], tsem.at[buf]).wait()
        pltpu.make_async_copy(
            ig_hbm.at[0, pl.ds(0, CH)], igbuf.at[buf], isem.at[buf]).wait()

    def _compute(chunk, buf):
        img = chunk // NCHUNK
        img_base = img * (ROWS * 16)

        @plsc.parallel_loop(0, NGRP, unroll=8)
        def _group(g):
            o = g * 16
            t = tbuf[buf, pl.ds(o, 16)]
            ig = igbuf[buf, pl.ds(o, 16)]
            t_eff = jnp.where(ig == 0, t, 0)
            pv = [pbuf[buf, c, pl.ds(o, 16)] for c in range(L)]
            sq = [v * v for v in pv]
            while len(sq) > 1:
                sq = [sq[2 * i] + sq[2 * i + 1] for i in range(len(sq) // 2)]
            n2 = sq[0]
            xi = lax.bitcast_convert_type(n2, jnp.int32)
            yi = jnp.int32(0x5F3759DF) - (xi >> 1)
            y = lax.bitcast_convert_type(yi, jnp.float32)
            for _ in range(2):
                y = y * (1.5 - 0.5 * n2 * y * y)
            inv = y - 1e-6 * (y * y)
            psq = n2 * inv * inv
            bvec = img_base + t_eff * 16 + lane
            for c in range(L):
                plsc.addupdate_scatter(
                    acc, [bvec + (c * NBIN * 16)], pv[c] * inv)
            plsc.addupdate_scatter(
                acc, [bvec + (L * NBIN * 16)], jnp.ones((16,), jnp.float32))
            plsc.addupdate_scatter(
                acc, [bvec + ((L + 1) * NBIN * 16)], psq)

    _start(0, 0)

    def _pair(i, carry):
        for b in range(2):
            cur = 2 * i + b
            nxt = cur + 1

            @pl.when(nxt < nchunks)
            def _prefetch():
                _start(nxt, 1 - b)

            _wait(b)
            _compute(cur, b)
        return carry

    lax.fori_loop(0, nchunks // 2, _pair, 0)
    pltpu.sync_copy(acc, out_hbm.at[wid])


def _tc_stage2(part_ref, out_ref):
    i = pl.program_id(0)

    @pl.when(i == 0)
    def _init():
        out_ref[...] = jnp.zeros_like(out_ref)

    x = part_ref[0]
    cols = [jnp.sum(x[c * NBIN + 1:c * NBIN + 1 + K, :], axis=1,
                    keepdims=True) for c in range(EXT)]
    acc = jnp.concatenate(cols, axis=1)
    sums = acc[:, :L]
    cnt = acc[:, L:L + 1]
    sq = acc[:, L + 1:L + 2]
    present = cnt > 0.0
    pm = present.astype(jnp.float32)
    nf = jnp.sum(pm)
    cnt_safe = jnp.maximum(cnt, 1.0)
    ssum = jnp.sum(sums * sums, axis=1, keepdims=True)
    mse = (sq - ssum / cnt_safe) / (L * cnt_safe)
    pull_sum = jnp.sum(jnp.where(present, mse, 0.0))
    s_col = jnp.sum(sums, axis=1, keepdims=True) / cnt_safe
    onehot_cnt = (lax.broadcasted_iota(jnp.int32, (1, EXT), 1) == L
                  ).astype(jnp.float32)
    cnt_row = lax.dot_general(
        onehot_cnt, acc, (((1,), (1,)), ((), ())),
        preferred_element_type=jnp.float32)
    ones_row = jnp.ones((1, L), jnp.float32)
    s_row = lax.dot_general(
        ones_row, sums / cnt_safe, (((1,), (1,)), ((), ())),
        preferred_element_type=jnp.float32)
    pm_row = (cnt_row > 0.0).astype(jnp.float32)
    ds = s_row - s_col
    push_raw = jnp.sum(pm * pm_row * jnp.exp(-(ds * ds)))
    eps = 1e-6
    pull = jnp.where(nf > 0.0, pull_sum / (nf + eps), 0.0)
    push = jnp.where(nf > 1.0, push_raw / ((nf - 1.0) * nf + eps), 0.0)
    out_ref[...] += jnp.reshape(pull + 0.1 * push, (1, 1))


@jax.jit
def kernel(pred, target, ignore_position):
    predr = pred.reshape(BS, L, P)
    tr = target.reshape(BS, P).astype(jnp.int32)
    igr = ignore_position.reshape(BS, P).astype(jnp.int32)

    mesh = plsc.VectorSubcoreMesh(core_axis_name="c", subcore_axis_name="s")
    stage1 = functools.partial(
        pl.kernel,
        mesh=mesh,
        out_type=jax.ShapeDtypeStruct((NW, BS * ROWS * 16), jnp.float32),
        scratch_types=[
            pltpu.VMEM((2, L, CH), jnp.float32),
            pltpu.VMEM((2, CH), jnp.int32),
            pltpu.VMEM((2, CH), jnp.int32),
            pltpu.VMEM((BS * ROWS * 16,), jnp.float32),
            pltpu.SemaphoreType.DMA((2,)),
            pltpu.SemaphoreType.DMA((2,)),
            pltpu.SemaphoreType.DMA((2,)),
        ],
        compiler_params=pltpu.CompilerParams(needs_layout_passes=False),
    )(_sc_stage1)
    partials = stage1(predr, tr, igr)

    part = (partials.reshape(NW, BS, ROWS, 16)
            .transpose(1, 2, 0, 3).reshape(BS, ROWS, NW * 16))
    out = pl.pallas_call(
        _tc_stage2,
        grid=(BS,),
        in_specs=[pl.BlockSpec((1, ROWS, NW * 16), lambda i: (i, 0, 0))],
        out_specs=pl.BlockSpec((1, 1), lambda i: (0, 0)),
        out_shape=jax.ShapeDtypeStruct((1, 1), jnp.float32),
        compiler_params=pltpu.CompilerParams(
            dimension_semantics=("arbitrary",)),
    )(part)
    return out[0, 0]


# speedup vs baseline: 1.4665x; 1.2604x over previous
import functools

import jax
import jax.numpy as jnp
from jax import lax
from jax.experimental import pallas as pl
from jax.experimental.pallas import tpu as pltpu
from jax.experimental.pallas import tpu_sc as plsc

K = 16
NBIN = 17
L = 32
P = 384 * 384
BS = 4
EXT = L + 2
NW = 32
PBTC = 9216
NTC = 8
P_TC = NTC * PBTC
PPW = (P - P_TC) // NW
NCHUNK = 3
CH = PPW // NCHUNK
NGRP = CH // 16
ROWS = EXT * NBIN


def _sc_stage1(pred_hbm, t_hbm, ig_hbm, out_hbm, pbuf, tbuf, igbuf, acc,
               psem, tsem, isem):
    wid = lax.axis_index("s") * 2 + lax.axis_index("c")
    lane = lax.broadcasted_iota(jnp.int32, (16,), 0)
    nchunks = BS * NCHUNK

    @plsc.parallel_loop(0, BS * ROWS, unroll=8)
    def _zero(i):
        acc[pl.ds(i * 16, 16)] = jnp.zeros((16,), jnp.float32)

    def _start(chunk, buf):
        img = chunk // NCHUNK
        base = P_TC + wid * PPW + (chunk % NCHUNK) * CH
        pltpu.make_async_copy(
            pred_hbm.at[img, :, pl.ds(base, CH)], pbuf.at[buf],
            psem.at[buf]).start()
        pltpu.make_async_copy(
            t_hbm.at[img, pl.ds(base, CH)], tbuf.at[buf],
            tsem.at[buf]).start()
        pltpu.make_async_copy(
            ig_hbm.at[img, pl.ds(base, CH)], igbuf.at[buf],
            isem.at[buf]).start()

    def _wait(buf):
        pltpu.make_async_copy(
            pred_hbm.at[0, :, pl.ds(0, CH)], pbuf.at[buf],
            psem.at[buf]).wait()
        pltpu.make_async_copy(
            t_hbm.at[0, pl.ds(0, CH)], tbuf.at[buf], tsem.at[buf]).wait()
        pltpu.make_async_copy(
            ig_hbm.at[0, pl.ds(0, CH)], igbuf.at[buf], isem.at[buf]).wait()

    def _compute(chunk, buf):
        img = chunk // NCHUNK
        img_base = img * (ROWS * 16)

        @plsc.parallel_loop(0, NGRP, unroll=4)
        def _group(g):
            o = g * 16
            t = tbuf[buf, pl.ds(o, 16)]
            ig = igbuf[buf, pl.ds(o, 16)]
            t_eff = jnp.where(ig == 0, t, 0)
            pv = [pbuf[buf, c, pl.ds(o, 16)] for c in range(L)]
            sq = [v * v for v in pv]
            while len(sq) > 1:
                sq = [sq[2 * i] + sq[2 * i + 1] for i in range(len(sq) // 2)]
            n2 = sq[0]
            xi = lax.bitcast_convert_type(n2, jnp.int32)
            yi = jnp.int32(0x5F3759DF) - (xi >> 1)
            y = lax.bitcast_convert_type(yi, jnp.float32)
            for _ in range(2):
                y = y * (1.5 - 0.5 * n2 * y * y)
            inv = y - 1e-6 * (y * y)
            psq = n2 * inv * inv
            bvec = img_base + t_eff * 16 + lane
            for c in range(L):
                plsc.addupdate_scatter(
                    acc, [bvec + (c * NBIN * 16)], pv[c] * inv)
            plsc.addupdate_scatter(
                acc, [bvec + (L * NBIN * 16)], jnp.ones((16,), jnp.float32))
            plsc.addupdate_scatter(
                acc, [bvec + ((L + 1) * NBIN * 16)], psq)

    _start(0, 0)

    def _pair(i, carry):
        for b in range(2):
            cur = 2 * i + b
            nxt = cur + 1

            @pl.when(nxt < nchunks)
            def _prefetch():
                _start(nxt, 1 - b)

            _wait(b)
            _compute(cur, b)
        return carry

    lax.fori_loop(0, nchunks // 2, _pair, 0)
    pltpu.sync_copy(acc, out_hbm.at[wid])


def _tc_partial(pred_ref, t_ref, ig_ref, out_ref):
    j = pl.program_id(1)

    @pl.when(j == 0)
    def _init():
        out_ref[...] = jnp.zeros_like(out_ref)

    p = pred_ref[0]
    t = t_ref[0, 0]
    ig = ig_ref[0, 0]

    n2 = jnp.sum(p * p, axis=0, keepdims=True)
    inv = 1.0 / (jnp.sqrt(n2) + 1e-6)
    pn = p * inv
    psq = n2 * (inv * inv)
    ones = jnp.ones((1, PBTC), jnp.float32)
    ext = jnp.concatenate([pn, ones, psq], axis=0)
    ids = lax.broadcasted_iota(jnp.int32, (K, PBTC), 0) + 1
    mf = ((t == ids) & (ig == 0)).astype(jnp.float32)
    out_ref[0] += lax.dot_general(
        mf, ext, (((1,), (1,)), ((), ())),
        preferred_element_type=jnp.float32)


def _tc_stage2(part_ref, tcp_ref, out_ref):
    i = pl.program_id(0)

    @pl.when(i == 0)
    def _init():
        out_ref[...] = jnp.zeros_like(out_ref)

    x = part_ref[0]
    cols = [jnp.sum(x[c * NBIN + 1:c * NBIN + 1 + K, :], axis=1,
                    keepdims=True) for c in range(EXT)]
    acc = jnp.concatenate(cols, axis=1) + tcp_ref[0]
    sums = acc[:, :L]
    cnt = acc[:, L:L + 1]
    sq = acc[:, L + 1:L + 2]
    present = cnt > 0.0
    pm = present.astype(jnp.float32)
    nf = jnp.sum(pm)
    cnt_safe = jnp.maximum(cnt, 1.0)
    ssum = jnp.sum(sums * sums, axis=1, keepdims=True)
    mse = (sq - ssum / cnt_safe) / (L * cnt_safe)
    pull_sum = jnp.sum(jnp.where(present, mse, 0.0))
    s_col = jnp.sum(sums, axis=1, keepdims=True) / cnt_safe
    onehot_cnt = (lax.broadcasted_iota(jnp.int32, (1, EXT), 1) == L
                  ).astype(jnp.float32)
    cnt_row = lax.dot_general(
        onehot_cnt, acc, (((1,), (1,)), ((), ())),
        preferred_element_type=jnp.float32)
    ones_row = jnp.ones((1, L), jnp.float32)
    s_row = lax.dot_general(
        ones_row, sums / cnt_safe, (((1,), (1,)), ((), ())),
        preferred_element_type=jnp.float32)
    pm_row = (cnt_row > 0.0).astype(jnp.float32)
    ds = s_row - s_col
    push_raw = jnp.sum(pm * pm_row * jnp.exp(-(ds * ds)))
    eps = 1e-6
    pull = jnp.where(nf > 0.0, pull_sum / (nf + eps), 0.0)
    push = jnp.where(nf > 1.0, push_raw / ((nf - 1.0) * nf + eps), 0.0)
    out_ref[...] += jnp.reshape(pull + 0.1 * push, (1, 1))


@jax.jit
def kernel(pred, target, ignore_position):
    predr = pred.reshape(BS, L, P)
    tr = target.reshape(BS, P).astype(jnp.int32)
    igr = ignore_position.reshape(BS, P).astype(jnp.int32)

    mesh = plsc.VectorSubcoreMesh(core_axis_name="c", subcore_axis_name="s")
    stage1 = functools.partial(
        pl.kernel,
        mesh=mesh,
        out_type=jax.ShapeDtypeStruct((NW, BS * ROWS * 16), jnp.float32),
        scratch_types=[
            pltpu.VMEM((2, L, CH), jnp.float32),
            pltpu.VMEM((2, CH), jnp.int32),
            pltpu.VMEM((2, CH), jnp.int32),
            pltpu.VMEM((BS * ROWS * 16,), jnp.float32),
            pltpu.SemaphoreType.DMA((2,)),
            pltpu.SemaphoreType.DMA((2,)),
            pltpu.SemaphoreType.DMA((2,)),
        ],
        compiler_params=pltpu.CompilerParams(needs_layout_passes=False),
    )(_sc_stage1)
    partials = stage1(predr, tr, igr)

    tr4 = tr.reshape(BS, P // PBTC, 1, PBTC)
    igr4 = igr.reshape(BS, P // PBTC, 1, PBTC)
    tc_part = pl.pallas_call(
        _tc_partial,
        grid=(BS, NTC),
        in_specs=[
            pl.BlockSpec((1, L, PBTC), lambda b, j: (b, 0, j)),
            pl.BlockSpec((1, 1, 1, PBTC), lambda b, j: (b, j, 0, 0)),
            pl.BlockSpec((1, 1, 1, PBTC), lambda b, j: (b, j, 0, 0)),
        ],
        out_specs=pl.BlockSpec((1, K, EXT), lambda b, j: (b, 0, 0)),
        out_shape=jax.ShapeDtypeStruct((BS, K, EXT), jnp.float32),
        compiler_params=pltpu.CompilerParams(
            dimension_semantics=("arbitrary", "arbitrary")),
    )(predr, tr4, igr4)

    part = (partials.reshape(NW, BS, ROWS, 16)
            .transpose(1, 2, 0, 3).reshape(BS, ROWS, NW * 16))
    out = pl.pallas_call(
        _tc_stage2,
        grid=(BS,),
        in_specs=[
            pl.BlockSpec((1, ROWS, NW * 16), lambda i: (i, 0, 0)),
            pl.BlockSpec((1, K, EXT), lambda i: (i, 0, 0)),
        ],
        out_specs=pl.BlockSpec((1, 1), lambda i: (0, 0)),
        out_shape=jax.ShapeDtypeStruct((1, 1), jnp.float32),
        compiler_params=pltpu.CompilerParams(
            dimension_semantics=("arbitrary",)),
    )(part, tc_part)
    return out[0, 0]
